# baseline (speedup 1.0000x reference)
import functools

import jax
import jax.numpy as jnp
from jax import lax
from jax.experimental import pallas as pl
from jax.experimental.pallas import tpu as pltpu
from jax.experimental.pallas import tpu_sc as plsc

N_NODES = 10000
D = 128
N_ACC = 10160
HIST_LO = 10000
HIST_ROWS = 80
NC = 2
NS = 16
NW = NC * NS
G = 128
CB = 16
C_CORE = (64, 96)


def _sc_segment_sum(x, rowm, colm, wm):
    C0, C1 = C_CORE
    NB0, NB1 = C0 // CB, C1 // CB
    RPT = 640
    TAIL = N_ACC - 15 * RPT
    mesh = plsc.VectorSubcoreMesh(core_axis_name="c", subcore_axis_name="s")

    @functools.partial(
        pl.kernel,
        out_type=[jax.ShapeDtypeStruct((N_ACC, D), jnp.float32)] * 2,
        mesh=mesh,
        scratch_types=[
            pltpu.VMEM((CB, G), jnp.int32),
            pltpu.VMEM((CB, G), jnp.int32),
            pltpu.VMEM((CB, G), jnp.float32),
            pltpu.VMEM((G, D), jnp.float32),
            pltpu.VMEM((G, D), jnp.float32),
            pltpu.VMEM((HIST_ROWS, D), jnp.float32),
            pltpu.VMEM((HIST_ROWS,), jnp.int32),
            pltpu.VMEM_SHARED((N_ACC, D), jnp.float32),
            pltpu.SemaphoreType.DMA,
            pltpu.SemaphoreType.DMA,
        ],
    )
    def k(x_hbm, row_hbm, col_hbm, w_hbm, out0, out1,
          col_v, row_v, w_v, rows_a, rows_b, hist_v, hidx_v, acc,
          sem_a, sem_b):
        cid = lax.axis_index("c")
        sid = lax.axis_index("s")
        chunk0 = jnp.where(cid == 0, sid * C0, NS * C0 + sid * C1)
        nb = jnp.where(cid == 0, NB0, NB1)

        zero16 = jnp.zeros((16,), jnp.float32)
        idx16 = lax.iota(jnp.int32, 16)

        def zrow(i, c):
            for t in range(D // 16):
                rows_a[i, pl.ds(t * 16, 16)] = zero16
            return c
        lax.fori_loop(0, G, zrow, 0)

        def zhist(i, c):
            for t in range(D // 16):
                hist_v[i, pl.ds(t * 16, 16)] = zero16
            return c
        lax.fori_loop(0, HIST_ROWS, zhist, 0)

        def whidx(g, c):
            hidx_v[pl.ds(g * 16, 16)] = idx16 + (g * 16 + HIST_LO)
            return c
        lax.fori_loop(0, HIST_ROWS // 16, whidx, 0)

        for kk in range(RPT // G):
            base = sid * RPT + kk * G
            if kk < RPT // G - 1:
                pltpu.sync_copy(rows_a, acc.at[pl.ds(base, G)])
            else:
                @pl.when(sid < NS - 1)
                def _():
                    pltpu.sync_copy(rows_a, acc.at[pl.ds(base, G)])

                @pl.when(sid == NS - 1)
                def _():
                    pltpu.sync_copy(rows_a.at[pl.ds(0, TAIL % G)],
                                    acc.at[pl.ds(base, TAIL % G)])
        plsc.subcore_barrier()

        def load_block(b):
            pltpu.sync_copy(col_hbm.at[pl.ds(chunk0 + b * CB, CB)], col_v)
            pltpu.sync_copy(row_hbm.at[pl.ds(chunk0 + b * CB, CB)], row_v)
            pltpu.sync_copy(w_hbm.at[pl.ds(chunk0 + b * CB, CB)], w_v)

        def process(jj, buf):
            def group(g, c2):
                w16 = w_v[jj, pl.ds(g * 16, 16)]
                row16 = row_v[jj, pl.ds(g * 16, 16)]
                for e in range(16):
                    eidx = g * 16 + e
                    wb = jnp.full((16,), w16[e], jnp.float32)
                    for t in range(D // 16):
                        buf[eidx, pl.ds(t * 16, 16)] = (
                            buf[eidx, pl.ds(t * 16, 16)] * wb)
                    r = row16[e]
                    hr = lax.shift_right_logical(r, 7)
                    soff = (lax.shift_right_logical(r, 4) & 7) * 16
                    oh = jnp.where(idx16 == (r & 15), 1.0, 0.0)
                    cur = hist_v[hr, pl.ds(soff, 16)]
                    hist_v[hr, pl.ds(soff, 16)] = cur + oh
                return c2
            lax.fori_loop(0, G // 16, group, 0)
            pltpu.sync_copy(buf, acc.at[row_v.at[jj]], add=True)

        load_block(0)
        pltpu.async_copy(x_hbm.at[col_v.at[0]], rows_a, sem_a)

        def block(b, c):
            def pair(t, c2):
                jj0 = 2 * t
                pltpu.async_copy(x_hbm.at[col_v.at[jj0 + 1]], rows_b, sem_b)
                pltpu.make_async_copy(x_hbm.at[col_v.at[jj0]], rows_a,
                                      sem_a).wait()
                process(jj0, rows_a)

                @pl.when(jj0 + 2 < CB)
                def _():
                    pltpu.async_copy(x_hbm.at[col_v.at[jj0 + 2]], rows_a,
                                     sem_a)
                pltpu.make_async_copy(x_hbm.at[col_v.at[jj0 + 1]], rows_b,
                                      sem_b).wait()
                process(jj0 + 1, rows_b)
                return c2
            lax.fori_loop(0, CB // 2, pair, 0)

            @pl.when(b + 1 < nb)
            def _():
                load_block(b + 1)
                pltpu.async_copy(x_hbm.at[col_v.at[0]], rows_a, sem_a)
            return c
        lax.fori_loop(0, nb, block, 0)

        pltpu.sync_copy(hist_v, acc.at[hidx_v], add=True)
        plsc.subcore_barrier()

        def wb(base, sz):
            pltpu.sync_copy(acc.at[pl.ds(base, sz)], rows_a.at[pl.ds(0, sz)])

            @pl.when(cid == 0)
            def _():
                pltpu.sync_copy(rows_a.at[pl.ds(0, sz)],
                                out0.at[pl.ds(base, sz)])

            @pl.when(cid == 1)
            def _():
                pltpu.sync_copy(rows_a.at[pl.ds(0, sz)],
                                out1.at[pl.ds(base, sz)])

        for kk in range(RPT // G):
            base2 = sid * RPT + kk * G
            if kk < RPT // G - 1:
                wb(base2, G)
            else:
                @pl.when(sid < NS - 1)
                def _():
                    wb(base2, G)

                @pl.when(sid == NS - 1)
                def _():
                    wb(base2, TAIL % G)

    return k(x, rowm, colm, wm)


def _tc_combine(x, a0, a1, h0, h1, W_self, W_nbr, b_self, b_nbr):
    R = 1000

    def body(x_ref, a0_ref, a1_ref, h0_ref, h1_ref,
             ws_ref, wn_ref, bs_ref, bn_ref, o_ref):
        feat = a0_ref[...] + a1_ref[...]
        cnt = h0_ref[...] + h1_ref[...]
        o_ref[...] = (
            jnp.dot(x_ref[...], ws_ref[...],
                    preferred_element_type=jnp.float32)
            + jnp.dot(feat, wn_ref[...], preferred_element_type=jnp.float32)
            + bs_ref[...] + cnt * bn_ref[...])

    return pl.pallas_call(
        body,
        grid=(N_NODES // R,),
        in_specs=[
            pl.BlockSpec((R, D), lambda i: (i, 0)),
            pl.BlockSpec((R, D), lambda i: (i, 0)),
            pl.BlockSpec((R, D), lambda i: (i, 0)),
            pl.BlockSpec((R, 1), lambda i: (i, 0)),
            pl.BlockSpec((R, 1), lambda i: (i, 0)),
            pl.BlockSpec((D, D), lambda i: (0, 0)),
            pl.BlockSpec((D, D), lambda i: (0, 0)),
            pl.BlockSpec((1, D), lambda i: (0, 0)),
            pl.BlockSpec((1, D), lambda i: (0, 0)),
        ],
        out_specs=pl.BlockSpec((R, D), lambda i: (i, 0)),
        out_shape=jax.ShapeDtypeStruct((N_NODES, D), jnp.float32),
    )(x, a0, a1, h0, h1, W_self, W_nbr,
      b_self.reshape(1, D), b_nbr.reshape(1, D))


def kernel(x, edge_index, edge_weights, W_self, b_self, W_nbr, b_nbr):
    x = x.astype(jnp.float32)
    row = edge_index[0].astype(jnp.int32)
    col = edge_index[1].astype(jnp.int32)
    w = edge_weights.astype(jnp.float32)

    E = row.shape[0]
    TOTC = NS * (C_CORE[0] + C_CORE[1])
    pad = TOTC * G - E
    pad_rows = (HIST_LO
                + (jnp.arange(pad, dtype=jnp.int32) % HIST_ROWS))
    rowm = jnp.concatenate([row, pad_rows]).reshape(TOTC, G)
    colm = jnp.concatenate([col, jnp.zeros((pad,), jnp.int32)]).reshape(
        TOTC, G)
    rep = jnp.arange(TOTC) % 8
    colm = colm + (N_NODES * rep[:, None]).astype(jnp.int32)
    wm = jnp.concatenate([w, jnp.zeros((pad,), jnp.float32)]).reshape(
        TOTC, G)

    a0, a1 = _sc_segment_sum(jnp.tile(x, (8, 1)), rowm, colm, wm)
    h0 = a0[HIST_LO:HIST_LO + HIST_ROWS].reshape(HIST_ROWS * D, 1)[:N_NODES]
    h1 = a1[HIST_LO:HIST_LO + HIST_ROWS].reshape(HIST_ROWS * D, 1)[:N_NODES]
    return _tc_combine(x, a0, a1, h0, h1, W_self, W_nbr, b_self, b_nbr)


# speedup vs baseline: 1.1999x; 1.1999x over previous
import functools

import jax
import jax.numpy as jnp
from jax import lax
from jax.experimental import pallas as pl
from jax.experimental.pallas import tpu as pltpu
from jax.experimental.pallas import tpu_sc as plsc

N_NODES = 10000
D = 128
N_ACC = 10160
HIST_LO = 10000
HIST_ROWS = 80
NC = 2
NS = 16
NW = NC * NS
G = 128
CB = 16
C_CORE = (96, 64)


def _sc_segment_sum(x, rowm, colm, wm):
    C0, C1 = C_CORE
    NB0, NB1 = C0 // CB, C1 // CB
    RPT = 640
    TAIL = N_ACC - 15 * RPT
    mesh = plsc.VectorSubcoreMesh(core_axis_name="c", subcore_axis_name="s")

    @functools.partial(
        pl.kernel,
        out_type=[jax.ShapeDtypeStruct((N_ACC, D), jnp.float32)] * 2,
        mesh=mesh,
        scratch_types=[
            pltpu.VMEM((CB, G), jnp.int32),
            pltpu.VMEM((CB, G), jnp.int32),
            pltpu.VMEM((CB, G), jnp.float32),
            pltpu.VMEM((G, D), jnp.float32),
            pltpu.VMEM((G, D), jnp.float32),
            pltpu.VMEM((HIST_ROWS, D), jnp.float32),
            pltpu.VMEM((HIST_ROWS,), jnp.int32),
            pltpu.VMEM_SHARED((N_ACC, D), jnp.float32),
            pltpu.SemaphoreType.DMA,
            pltpu.SemaphoreType.DMA,
        ],
    )
    def k(x_hbm, row_hbm, col_hbm, w_hbm, out0, out1,
          col_v, row_v, w_v, rows_a, rows_b, hist_v, hidx_v, acc,
          sem_a, sem_b):
        cid = lax.axis_index("c")
        sid = lax.axis_index("s")
        chunk0 = jnp.where(cid == 0, sid * C0, NS * C0 + sid * C1)
        nb = jnp.where(cid == 0, NB0, NB1)

        zero16 = jnp.zeros((16,), jnp.float32)
        idx16 = lax.iota(jnp.int32, 16)

        def zrow(i, c):
            for t in range(D // 16):
                rows_a[i, pl.ds(t * 16, 16)] = zero16
            return c
        lax.fori_loop(0, G, zrow, 0)

        def zhist(i, c):
            for t in range(D // 16):
                hist_v[i, pl.ds(t * 16, 16)] = zero16
            return c
        lax.fori_loop(0, HIST_ROWS, zhist, 0)

        def whidx(g, c):
            hidx_v[pl.ds(g * 16, 16)] = idx16 + (g * 16 + HIST_LO)
            return c
        lax.fori_loop(0, HIST_ROWS // 16, whidx, 0)

        for kk in range(RPT // G):
            base = sid * RPT + kk * G
            if kk < RPT // G - 1:
                pltpu.sync_copy(rows_a, acc.at[pl.ds(base, G)])
            else:
                @pl.when(sid < NS - 1)
                def _():
                    pltpu.sync_copy(rows_a, acc.at[pl.ds(base, G)])

                @pl.when(sid == NS - 1)
                def _():
                    pltpu.sync_copy(rows_a.at[pl.ds(0, TAIL % G)],
                                    acc.at[pl.ds(base, TAIL % G)])
        plsc.subcore_barrier()

        def load_block(b):
            pltpu.sync_copy(col_hbm.at[pl.ds(chunk0 + b * CB, CB)], col_v)
            pltpu.sync_copy(row_hbm.at[pl.ds(chunk0 + b * CB, CB)], row_v)
            pltpu.sync_copy(w_hbm.at[pl.ds(chunk0 + b * CB, CB)], w_v)

        def process(jj, buf):
            def group(g, c2):
                w16 = w_v[jj, pl.ds(g * 16, 16)]
                row16 = row_v[jj, pl.ds(g * 16, 16)]
                for e in range(16):
                    eidx = g * 16 + e
                    wb = jnp.full((16,), w16[e], jnp.float32)
                    for t in range(D // 16):
                        buf[eidx, pl.ds(t * 16, 16)] = (
                            buf[eidx, pl.ds(t * 16, 16)] * wb)
                    r = row16[e]
                    hr = lax.shift_right_logical(r, 7)
                    soff = (lax.shift_right_logical(r, 4) & 7) * 16
                    oh = jnp.where(idx16 == (r & 15), 1.0, 0.0)
                    cur = hist_v[hr, pl.ds(soff, 16)]
                    hist_v[hr, pl.ds(soff, 16)] = cur + oh
                return c2
            lax.fori_loop(0, G // 16, group, 0)
            pltpu.sync_copy(buf, acc.at[row_v.at[jj]], add=True)

        load_block(0)
        pltpu.async_copy(x_hbm.at[col_v.at[0]], rows_a, sem_a)

        def block(b, c):
            def pair(t, c2):
                jj0 = 2 * t
                pltpu.async_copy(x_hbm.at[col_v.at[jj0 + 1]], rows_b, sem_b)
                pltpu.make_async_copy(x_hbm.at[col_v.at[jj0]], rows_a,
                                      sem_a).wait()
                process(jj0, rows_a)

                @pl.when(jj0 + 2 < CB)
                def _():
                    pltpu.async_copy(x_hbm.at[col_v.at[jj0 + 2]], rows_a,
                                     sem_a)
                pltpu.make_async_copy(x_hbm.at[col_v.at[jj0 + 1]], rows_b,
                                      sem_b).wait()
                process(jj0 + 1, rows_b)
                return c2
            lax.fori_loop(0, CB // 2, pair, 0)

            @pl.when(b + 1 < nb)
            def _():
                load_block(b + 1)
                pltpu.async_copy(x_hbm.at[col_v.at[0]], rows_a, sem_a)
            return c
        lax.fori_loop(0, nb, block, 0)

        pltpu.sync_copy(hist_v, acc.at[hidx_v], add=True)
        plsc.subcore_barrier()

        def wb(base, sz):
            pltpu.sync_copy(acc.at[pl.ds(base, sz)], rows_a.at[pl.ds(0, sz)])

            @pl.when(cid == 0)
            def _():
                pltpu.sync_copy(rows_a.at[pl.ds(0, sz)],
                                out0.at[pl.ds(base, sz)])

            @pl.when(cid == 1)
            def _():
                pltpu.sync_copy(rows_a.at[pl.ds(0, sz)],
                                out1.at[pl.ds(base, sz)])

        for kk in range(RPT // G):
            base2 = sid * RPT + kk * G
            if kk < RPT // G - 1:
                wb(base2, G)
            else:
                @pl.when(sid < NS - 1)
                def _():
                    wb(base2, G)

                @pl.when(sid == NS - 1)
                def _():
                    wb(base2, TAIL % G)

    return k(x, rowm, colm, wm)


def _tc_combine(x, a0, a1, h0, h1, W_self, W_nbr, b_self, b_nbr):
    R = 1000

    def body(x_ref, a0_ref, a1_ref, h0_ref, h1_ref,
             ws_ref, wn_ref, bs_ref, bn_ref, o_ref):
        feat = a0_ref[...] + a1_ref[...]
        cnt = h0_ref[...] + h1_ref[...]
        o_ref[...] = (
            jnp.dot(x_ref[...], ws_ref[...],
                    preferred_element_type=jnp.float32)
            + jnp.dot(feat, wn_ref[...], preferred_element_type=jnp.float32)
            + bs_ref[...] + cnt * bn_ref[...])

    return pl.pallas_call(
        body,
        grid=(N_NODES // R,),
        in_specs=[
            pl.BlockSpec((R, D), lambda i: (i, 0)),
            pl.BlockSpec((R, D), lambda i: (i, 0)),
            pl.BlockSpec((R, D), lambda i: (i, 0)),
            pl.BlockSpec((R, 1), lambda i: (i, 0)),
            pl.BlockSpec((R, 1), lambda i: (i, 0)),
            pl.BlockSpec((D, D), lambda i: (0, 0)),
            pl.BlockSpec((D, D), lambda i: (0, 0)),
            pl.BlockSpec((1, D), lambda i: (0, 0)),
            pl.BlockSpec((1, D), lambda i: (0, 0)),
        ],
        out_specs=pl.BlockSpec((R, D), lambda i: (i, 0)),
        out_shape=jax.ShapeDtypeStruct((N_NODES, D), jnp.float32),
    )(x, a0, a1, h0, h1, W_self, W_nbr,
      b_self.reshape(1, D), b_nbr.reshape(1, D))


def kernel(x, edge_index, edge_weights, W_self, b_self, W_nbr, b_nbr):
    x = x.astype(jnp.float32)
    row = edge_index[0].astype(jnp.int32)
    col = edge_index[1].astype(jnp.int32)
    w = edge_weights.astype(jnp.float32)

    E = row.shape[0]
    TOTC = NS * (C_CORE[0] + C_CORE[1])
    pad = TOTC * G - E
    pad_rows = (HIST_LO
                + (jnp.arange(pad, dtype=jnp.int32) % HIST_ROWS))
    rowm = jnp.concatenate([row, pad_rows]).reshape(TOTC, G)
    colm = jnp.concatenate([col, jnp.zeros((pad,), jnp.int32)]).reshape(
        TOTC, G)
    rep = jnp.arange(TOTC) % 8
    colm = colm + (N_NODES * rep[:, None]).astype(jnp.int32)
    wm = jnp.concatenate([w, jnp.zeros((pad,), jnp.float32)]).reshape(
        TOTC, G)

    a0, a1 = _sc_segment_sum(jnp.tile(x, (8, 1)), rowm, colm, wm)
    h0 = a0[HIST_LO:HIST_LO + HIST_ROWS].reshape(HIST_ROWS * D, 1)[:N_NODES]
    h1 = a1[HIST_LO:HIST_LO + HIST_ROWS].reshape(HIST_ROWS * D, 1)[:N_NODES]
    return _tc_combine(x, a0, a1, h0, h1, W_self, W_nbr, b_self, b_nbr)


# speedup vs baseline: 1.2267x; 1.0223x over previous
import functools

import jax
import jax.numpy as jnp
from jax import lax
from jax.experimental import pallas as pl
from jax.experimental.pallas import tpu as pltpu
from jax.experimental.pallas import tpu_sc as plsc

N_NODES = 10000
D = 128
N_ACC = 10160
HIST_LO = 10000
HIST_ROWS = 80
NC = 2
NS = 16
NW = NC * NS
G = 128
CB = 8
C_CORE = (104, 56)


def _sc_segment_sum(x, rowm, colm, wm):
    C0, C1 = C_CORE
    NB0, NB1 = C0 // CB, C1 // CB
    RPT = 640
    TAIL = N_ACC - 15 * RPT
    mesh = plsc.VectorSubcoreMesh(core_axis_name="c", subcore_axis_name="s")

    @functools.partial(
        pl.kernel,
        out_type=[jax.ShapeDtypeStruct((N_ACC, D), jnp.float32)] * 2,
        mesh=mesh,
        scratch_types=[
            pltpu.VMEM((CB, G), jnp.int32),
            pltpu.VMEM((CB, G), jnp.int32),
            pltpu.VMEM((CB, G), jnp.float32),
            pltpu.VMEM((G, D), jnp.float32),
            pltpu.VMEM((G, D), jnp.float32),
            pltpu.VMEM((HIST_ROWS, D), jnp.float32),
            pltpu.VMEM((HIST_ROWS,), jnp.int32),
            pltpu.VMEM_SHARED((N_ACC, D), jnp.float32),
            pltpu.SemaphoreType.DMA,
            pltpu.SemaphoreType.DMA,
        ],
    )
    def k(x_hbm, row_hbm, col_hbm, w_hbm, out0, out1,
          col_v, row_v, w_v, rows_a, rows_b, hist_v, hidx_v, acc,
          sem_a, sem_b):
        cid = lax.axis_index("c")
        sid = lax.axis_index("s")
        chunk0 = jnp.where(cid == 0, sid * C0, NS * C0 + sid * C1)
        nb = jnp.where(cid == 0, NB0, NB1)

        zero16 = jnp.zeros((16,), jnp.float32)
        idx16 = lax.iota(jnp.int32, 16)

        def zrow(i, c):
            for t in range(D // 16):
                rows_a[i, pl.ds(t * 16, 16)] = zero16
            return c
        lax.fori_loop(0, G, zrow, 0)

        def zhist(i, c):
            for t in range(D // 16):
                hist_v[i, pl.ds(t * 16, 16)] = zero16
            return c
        lax.fori_loop(0, HIST_ROWS, zhist, 0)

        def whidx(g, c):
            hidx_v[pl.ds(g * 16, 16)] = idx16 + (g * 16 + HIST_LO)
            return c
        lax.fori_loop(0, HIST_ROWS // 16, whidx, 0)

        for kk in range(RPT // G):
            base = sid * RPT + kk * G
            if kk < RPT // G - 1:
                pltpu.sync_copy(rows_a, acc.at[pl.ds(base, G)])
            else:
                @pl.when(sid < NS - 1)
                def _():
                    pltpu.sync_copy(rows_a, acc.at[pl.ds(base, G)])

                @pl.when(sid == NS - 1)
                def _():
                    pltpu.sync_copy(rows_a.at[pl.ds(0, TAIL % G)],
                                    acc.at[pl.ds(base, TAIL % G)])
        plsc.subcore_barrier()

        def load_block(b):
            pltpu.sync_copy(col_hbm.at[pl.ds(chunk0 + b * CB, CB)], col_v)
            pltpu.sync_copy(row_hbm.at[pl.ds(chunk0 + b * CB, CB)], row_v)
            pltpu.sync_copy(w_hbm.at[pl.ds(chunk0 + b * CB, CB)], w_v)

        def process(jj, buf):
            def group(g, c2):
                w16 = w_v[jj, pl.ds(g * 16, 16)]
                row16 = row_v[jj, pl.ds(g * 16, 16)]
                for e in range(16):
                    eidx = g * 16 + e
                    wb = jnp.full((16,), w16[e], jnp.float32)
                    for t in range(D // 16):
                        buf[eidx, pl.ds(t * 16, 16)] = (
                            buf[eidx, pl.ds(t * 16, 16)] * wb)
                    r = row16[e]
                    hr = lax.shift_right_logical(r, 7)
                    soff = (lax.shift_right_logical(r, 4) & 7) * 16
                    oh = jnp.where(idx16 == (r & 15), 1.0, 0.0)
                    cur = hist_v[hr, pl.ds(soff, 16)]
                    hist_v[hr, pl.ds(soff, 16)] = cur + oh
                return c2
            lax.fori_loop(0, G // 16, group, 0)
            pltpu.sync_copy(buf, acc.at[row_v.at[jj]], add=True)

        load_block(0)
        pltpu.async_copy(x_hbm.at[col_v.at[0]], rows_a, sem_a)

        def block(b, c):
            def pair(t, c2):
                jj0 = 2 * t
                pltpu.async_copy(x_hbm.at[col_v.at[jj0 + 1]], rows_b, sem_b)
                pltpu.make_async_copy(x_hbm.at[col_v.at[jj0]], rows_a,
                                      sem_a).wait()
                process(jj0, rows_a)

                @pl.when(jj0 + 2 < CB)
                def _():
                    pltpu.async_copy(x_hbm.at[col_v.at[jj0 + 2]], rows_a,
                                     sem_a)
                pltpu.make_async_copy(x_hbm.at[col_v.at[jj0 + 1]], rows_b,
                                      sem_b).wait()
                process(jj0 + 1, rows_b)
                return c2
            lax.fori_loop(0, CB // 2, pair, 0)

            @pl.when(b + 1 < nb)
            def _():
                load_block(b + 1)
                pltpu.async_copy(x_hbm.at[col_v.at[0]], rows_a, sem_a)
            return c
        lax.fori_loop(0, nb, block, 0)

        pltpu.sync_copy(hist_v, acc.at[hidx_v], add=True)
        plsc.subcore_barrier()

        def wb(base, sz):
            pltpu.sync_copy(acc.at[pl.ds(base, sz)], rows_a.at[pl.ds(0, sz)])

            @pl.when(cid == 0)
            def _():
                pltpu.sync_copy(rows_a.at[pl.ds(0, sz)],
                                out0.at[pl.ds(base, sz)])

            @pl.when(cid == 1)
            def _():
                pltpu.sync_copy(rows_a.at[pl.ds(0, sz)],
                                out1.at[pl.ds(base, sz)])

        for kk in range(RPT // G):
            base2 = sid * RPT + kk * G
            if kk < RPT // G - 1:
                wb(base2, G)
            else:
                @pl.when(sid < NS - 1)
                def _():
                    wb(base2, G)

                @pl.when(sid == NS - 1)
                def _():
                    wb(base2, TAIL % G)

    return k(x, rowm, colm, wm)


def _tc_combine(x, a0, a1, h0, h1, W_self, W_nbr, b_self, b_nbr):
    R = 1000

    def body(x_ref, a0_ref, a1_ref, h0_ref, h1_ref,
             ws_ref, wn_ref, bs_ref, bn_ref, o_ref):
        feat = a0_ref[...] + a1_ref[...]
        cnt = h0_ref[...] + h1_ref[...]
        o_ref[...] = (
            jnp.dot(x_ref[...], ws_ref[...],
                    preferred_element_type=jnp.float32)
            + jnp.dot(feat, wn_ref[...], preferred_element_type=jnp.float32)
            + bs_ref[...] + cnt * bn_ref[...])

    return pl.pallas_call(
        body,
        grid=(N_NODES // R,),
        in_specs=[
            pl.BlockSpec((R, D), lambda i: (i, 0)),
            pl.BlockSpec((R, D), lambda i: (i, 0)),
            pl.BlockSpec((R, D), lambda i: (i, 0)),
            pl.BlockSpec((R, 1), lambda i: (i, 0)),
            pl.BlockSpec((R, 1), lambda i: (i, 0)),
            pl.BlockSpec((D, D), lambda i: (0, 0)),
            pl.BlockSpec((D, D), lambda i: (0, 0)),
            pl.BlockSpec((1, D), lambda i: (0, 0)),
            pl.BlockSpec((1, D), lambda i: (0, 0)),
        ],
        out_specs=pl.BlockSpec((R, D), lambda i: (i, 0)),
        out_shape=jax.ShapeDtypeStruct((N_NODES, D), jnp.float32),
    )(x, a0, a1, h0, h1, W_self, W_nbr,
      b_self.reshape(1, D), b_nbr.reshape(1, D))


def kernel(x, edge_index, edge_weights, W_self, b_self, W_nbr, b_nbr):
    x = x.astype(jnp.float32)
    row = edge_index[0].astype(jnp.int32)
    col = edge_index[1].astype(jnp.int32)
    w = edge_weights.astype(jnp.float32)

    E = row.shape[0]
    TOTC = NS * (C_CORE[0] + C_CORE[1])
    pad = TOTC * G - E
    pad_rows = (HIST_LO
                + (jnp.arange(pad, dtype=jnp.int32) % HIST_ROWS))
    rowm = jnp.concatenate([row, pad_rows]).reshape(TOTC, G)
    colm = jnp.concatenate([col, jnp.zeros((pad,), jnp.int32)]).reshape(
        TOTC, G)
    rep = jnp.arange(TOTC) % 8
    colm = colm + (N_NODES * rep[:, None]).astype(jnp.int32)
    wm = jnp.concatenate([w, jnp.zeros((pad,), jnp.float32)]).reshape(
        TOTC, G)

    a0, a1 = _sc_segment_sum(jnp.tile(x, (8, 1)), rowm, colm, wm)
    h0 = a0[HIST_LO:HIST_LO + HIST_ROWS].reshape(HIST_ROWS * D, 1)[:N_NODES]
    h1 = a1[HIST_LO:HIST_LO + HIST_ROWS].reshape(HIST_ROWS * D, 1)[:N_NODES]
    return _tc_combine(x, a0, a1, h0, h1, W_self, W_nbr, b_self, b_nbr)
